# baseline (speedup 1.0000x reference)
import functools

import jax
import jax.numpy as jnp
from jax import lax
from jax.experimental import pallas as pl
from jax.experimental.pallas import tpu as pltpu
from jax.experimental.pallas import tpu_sc as plsc

_N = 10000
_E = 320000
_F = 128
_G = 64

_NCORES = 2
_NSUB = 16
_NTILES = _NCORES * _NSUB
_CH = 128
_K0 = 100
_K1 = 57
_IB = 45
_TRASH = _N
_RPT = 624
_REM = _N - _RPT * _NSUB
_ZR = 16


def _sc_aggregate(h, idx0, idx1):
  mesh = plsc.VectorSubcoreMesh(core_axis_name="c", subcore_axis_name="s")

  @functools.partial(
      pl.kernel,
      out_type=jax.ShapeDtypeStruct((_NCORES, _N, _F), jnp.float32),
      mesh=mesh,
      scratch_types=[
          pltpu.VMEM((_IB, 2, _CH), jnp.int32),
          pltpu.VMEM((_CH, _F), jnp.float32),
          pltpu.VMEM((_CH, _F), jnp.float32),
          pltpu.VMEM((_ZR, _F), jnp.float32),
          pltpu.VMEM_SHARED((_N + 8, _F), jnp.float32),
          pltpu.SemaphoreType.DMA,
          pltpu.SemaphoreType.DMA,
      ],
  )
  def agg(h_hbm, idx0_hbm, idx1_hbm, out_hbm, ibuf, bufa, bufb, zbuf, part,
          gsa, gsb):
    cid = lax.axis_index("c")
    sid = lax.axis_index("s")
    tid = cid * _NSUB + sid
    zvec = jnp.zeros((16,), jnp.float32)

    def zrow(i, carry):
      for j in range(_F // 16):
        zbuf[i, pl.ds(j * 16, 16)] = zvec
      return carry

    lax.fori_loop(0, _ZR, zrow, 0)

    def zcopy(j, carry):
      pltpu.sync_copy(zbuf, part.at[pl.ds(sid * _RPT + j * _ZR, _ZR)])
      return carry

    lax.fori_loop(0, _RPT // _ZR, zcopy, 0)
    @pl.when(sid == 0)
    def _():
      pltpu.sync_copy(zbuf.at[pl.ds(0, _REM)],
                      part.at[pl.ds(_RPT * _NSUB, _REM)])
    plsc.subcore_barrier()

    def fire(k, buf, sem):
      pltpu.async_copy(h_hbm.at[ibuf.at[k, 0]], buf, sem)

    def drain(k, buf, sem):
      pltpu.make_async_copy(h_hbm.at[ibuf.at[k, 0]], buf, sem).wait()

    def scat(k, buf):
      pltpu.sync_copy(buf, part.at[ibuf.at[k, 1]], add=True)

    def run_block(idx_hbm, base, n):
      pltpu.sync_copy(idx_hbm.at[sid, pl.ds(base, n)], ibuf.at[pl.ds(0, n)])
      fire(0, bufa, gsa)

      def pair(i, carry):
        k = 2 * i
        fire(k + 1, bufb, gsb)
        drain(k, bufa, gsa)
        scat(k, bufa)
        fire(k + 2, bufa, gsa)
        drain(k + 1, bufb, gsb)
        scat(k + 1, bufb)
        return carry

      m = (n - 1) // 2
      lax.fori_loop(0, m, pair, 0)
      if n % 2 == 1:
        drain(n - 1, bufa, gsa)
        scat(n - 1, bufa)
      else:
        fire(n - 1, bufb, gsb)
        drain(n - 2, bufa, gsa)
        scat(n - 2, bufa)
        drain(n - 1, bufb, gsb)
        scat(n - 1, bufb)

    def run_core(idx_hbm, k):
      for base in range(0, k, _IB):
        run_block(idx_hbm, base, min(_IB, k - base))

    @pl.when(cid == 0)
    def _():
      run_core(idx0_hbm, _K0)

    @pl.when(cid == 1)
    def _():
      run_core(idx1_hbm, _K1)
    plsc.subcore_barrier()
    pltpu.sync_copy(part.at[pl.ds(sid * _RPT, _RPT)],
                    out_hbm.at[cid, pl.ds(sid * _RPT, _RPT)])
    @pl.when(sid == 0)
    def _():
      pltpu.sync_copy(part.at[pl.ds(_RPT * _NSUB, _REM)],
                      out_hbm.at[cid, pl.ds(_RPT * _NSUB, _REM)])

  return agg(h, idx0, idx1)


def _tc_layer(h, parts, W1, b1, W2, b2, g, bt):
  def body(h_ref, p_ref, w1_ref, b1_ref, w2_ref, b2_ref, g_ref, bt_ref, o_ref):
    h2 = h_ref[...] + p_ref[0] + p_ref[1]
    z = jnp.dot(h2, w1_ref[...], preferred_element_type=jnp.float32) + b1_ref[...]
    z = jnp.maximum(z, 0.0)
    z = jnp.dot(z, w2_ref[...], preferred_element_type=jnp.float32) + b2_ref[...]
    mean = jnp.mean(z, axis=0, keepdims=True)
    d = z - mean
    var = jnp.mean(d * d, axis=0, keepdims=True)
    scale = g_ref[...] * lax.rsqrt(var + 1e-5)
    o_ref[...] = jnp.maximum(d * scale + bt_ref[...], 0.0)

  return pl.pallas_call(
      body,
      out_shape=jax.ShapeDtypeStruct((_N, _F), jnp.float32),
  )(h, parts, W1, b1.reshape(1, _F), W2, b2.reshape(1, _F),
    g.reshape(1, _F), bt.reshape(1, _F))


def _tc_final(h, parts, W1, b1, W2, b2, g, bt, batch_idx, Wp, bp):
  def body(h_ref, p_ref, w1_ref, b1_ref, w2_ref, b2_ref, g_ref, bt_ref,
           bi_ref, wp_ref, bp_ref, o_ref):
    h2 = h_ref[...] + p_ref[0] + p_ref[1]
    z = jnp.dot(h2, w1_ref[...], preferred_element_type=jnp.float32) + b1_ref[...]
    z = jnp.maximum(z, 0.0)
    z = jnp.dot(z, w2_ref[...], preferred_element_type=jnp.float32) + b2_ref[...]
    mean = jnp.mean(z, axis=0, keepdims=True)
    d = z - mean
    var = jnp.mean(d * d, axis=0, keepdims=True)
    scale = g_ref[...] * lax.rsqrt(var + 1e-5)
    z = jnp.maximum(d * scale + bt_ref[...], 0.0)
    onehot = (bi_ref[...] == lax.broadcasted_iota(jnp.int32, (1, _G), 1)
              ).astype(jnp.float32)
    sums = lax.dot_general(onehot, z, (((0,), (0,)), ((), ())),
                           preferred_element_type=jnp.float32)
    cnts = jnp.sum(onehot, axis=0, keepdims=True)
    pooled = sums * (1.0 / jnp.maximum(cnts, 1.0)).T
    out = jnp.dot(pooled, wp_ref[...], preferred_element_type=jnp.float32)
    o_ref[...] = jnp.maximum(out + bp_ref[...], 0.0)

  return pl.pallas_call(
      body,
      out_shape=jax.ShapeDtypeStruct((_G, _G), jnp.float32),
  )(h, parts, W1, b1.reshape(1, _F), W2, b2.reshape(1, _F),
    g.reshape(1, _F), bt.reshape(1, _F), batch_idx.reshape(_N, 1),
    Wp, bp.reshape(1, _G))


def kernel(x, edge_index, batch_idx, W1_0, b1_0, W2_0, b2_0, g_0, bt_0,
           W1_1, b1_1, W2_1, b2_1, g_1, bt_1, W1_2, b1_2, W2_2, b2_2, g_2,
           bt_2, Wp, bp):
  src = edge_index[0]
  dst = edge_index[1]
  e0 = _NSUB * _K0 * _CH
  cap1 = _NSUB * _K1 * _CH
  pad = e0 + cap1 - _E

  def mk(v, fill, k, lo, hi, padn):
    vv = jnp.concatenate([v[lo:hi], jnp.full((padn,), fill, jnp.int32)]) \
        if padn else v[lo:hi]
    return vv.reshape(_NSUB, k, _CH)

  idx0 = jnp.stack([mk(src, 0, _K0, 0, e0, 0),
                    mk(dst, _TRASH, _K0, 0, e0, 0)], axis=2)
  idx1 = jnp.stack([mk(src, 0, _K1, e0, _E, pad),
                    mk(dst, _TRASH, _K1, e0, _E, pad)], axis=2)

  h = x
  parts = _sc_aggregate(h, idx0, idx1)
  h = _tc_layer(h, parts, W1_0, b1_0, W2_0, b2_0, g_0, bt_0)
  parts = _sc_aggregate(h, idx0, idx1)
  h = _tc_layer(h, parts, W1_1, b1_1, W2_1, b2_1, g_1, bt_1)
  parts = _sc_aggregate(h, idx0, idx1)
  return _tc_final(h, parts, W1_2, b1_2, W2_2, b2_2, g_2, bt_2,
                   batch_idx, Wp, bp)


# speedup vs baseline: 1.0299x; 1.0299x over previous
import functools

import jax
import jax.numpy as jnp
from jax import lax
from jax.experimental import pallas as pl
from jax.experimental.pallas import tpu as pltpu
from jax.experimental.pallas import tpu_sc as plsc

_N = 10000
_E = 320000
_F = 128
_G = 64

_NCORES = 2
_NSUB = 16
_NTILES = _NCORES * _NSUB
_CH = 128
_K0 = 108
_K1 = 49
_IB = 45
_TRASH = _N
_RPT = 624
_REM = _N - _RPT * _NSUB
_ZR = 16


def _sc_aggregate(h, idx0, idx1):
  mesh = plsc.VectorSubcoreMesh(core_axis_name="c", subcore_axis_name="s")

  @functools.partial(
      pl.kernel,
      out_type=jax.ShapeDtypeStruct((_NCORES, _N, _F), jnp.float32),
      mesh=mesh,
      scratch_types=[
          pltpu.VMEM((_IB, 2, _CH), jnp.int32),
          pltpu.VMEM((_CH, _F), jnp.float32),
          pltpu.VMEM((_CH, _F), jnp.float32),
          pltpu.VMEM((_ZR, _F), jnp.float32),
          pltpu.VMEM_SHARED((_N + 8, _F), jnp.float32),
          pltpu.SemaphoreType.DMA,
          pltpu.SemaphoreType.DMA,
      ],
  )
  def agg(h_hbm, idx0_hbm, idx1_hbm, out_hbm, ibuf, bufa, bufb, zbuf, part,
          gsa, gsb):
    cid = lax.axis_index("c")
    sid = lax.axis_index("s")
    tid = cid * _NSUB + sid
    zvec = jnp.zeros((16,), jnp.float32)

    def zrow(i, carry):
      for j in range(_F // 16):
        zbuf[i, pl.ds(j * 16, 16)] = zvec
      return carry

    lax.fori_loop(0, _ZR, zrow, 0)

    def zcopy(j, carry):
      pltpu.sync_copy(zbuf, part.at[pl.ds(sid * _RPT + j * _ZR, _ZR)])
      return carry

    lax.fori_loop(0, _RPT // _ZR, zcopy, 0)
    @pl.when(sid == 0)
    def _():
      pltpu.sync_copy(zbuf.at[pl.ds(0, _REM)],
                      part.at[pl.ds(_RPT * _NSUB, _REM)])
    plsc.subcore_barrier()

    def fire(k, buf, sem):
      pltpu.async_copy(h_hbm.at[ibuf.at[k, 0]], buf, sem)

    def drain(k, buf, sem):
      pltpu.make_async_copy(h_hbm.at[ibuf.at[k, 0]], buf, sem).wait()

    def scat(k, buf):
      pltpu.sync_copy(buf, part.at[ibuf.at[k, 1]], add=True)

    def run_block(idx_hbm, base, n):
      pltpu.sync_copy(idx_hbm.at[sid, pl.ds(base, n)], ibuf.at[pl.ds(0, n)])
      fire(0, bufa, gsa)

      def pair(i, carry):
        k = 2 * i
        fire(k + 1, bufb, gsb)
        drain(k, bufa, gsa)
        scat(k, bufa)
        fire(k + 2, bufa, gsa)
        drain(k + 1, bufb, gsb)
        scat(k + 1, bufb)
        return carry

      m = (n - 1) // 2
      lax.fori_loop(0, m, pair, 0)
      if n % 2 == 1:
        drain(n - 1, bufa, gsa)
        scat(n - 1, bufa)
      else:
        fire(n - 1, bufb, gsb)
        drain(n - 2, bufa, gsa)
        scat(n - 2, bufa)
        drain(n - 1, bufb, gsb)
        scat(n - 1, bufb)

    def run_core(idx_hbm, k):
      for base in range(0, k, _IB):
        run_block(idx_hbm, base, min(_IB, k - base))

    @pl.when(cid == 0)
    def _():
      run_core(idx0_hbm, _K0)

    @pl.when(cid == 1)
    def _():
      run_core(idx1_hbm, _K1)
    plsc.subcore_barrier()
    pltpu.sync_copy(part.at[pl.ds(sid * _RPT, _RPT)],
                    out_hbm.at[cid, pl.ds(sid * _RPT, _RPT)])
    @pl.when(sid == 0)
    def _():
      pltpu.sync_copy(part.at[pl.ds(_RPT * _NSUB, _REM)],
                      out_hbm.at[cid, pl.ds(_RPT * _NSUB, _REM)])

  return agg(h, idx0, idx1)


def _tc_layer(h, parts, W1, b1, W2, b2, g, bt):
  def body(h_ref, p_ref, w1_ref, b1_ref, w2_ref, b2_ref, g_ref, bt_ref, o_ref):
    h2 = h_ref[...] + p_ref[0] + p_ref[1]
    z = jnp.dot(h2, w1_ref[...], preferred_element_type=jnp.float32) + b1_ref[...]
    z = jnp.maximum(z, 0.0)
    z = jnp.dot(z, w2_ref[...], preferred_element_type=jnp.float32) + b2_ref[...]
    mean = jnp.mean(z, axis=0, keepdims=True)
    d = z - mean
    var = jnp.mean(d * d, axis=0, keepdims=True)
    scale = g_ref[...] * lax.rsqrt(var + 1e-5)
    o_ref[...] = jnp.maximum(d * scale + bt_ref[...], 0.0)

  return pl.pallas_call(
      body,
      out_shape=jax.ShapeDtypeStruct((_N, _F), jnp.float32),
  )(h, parts, W1, b1.reshape(1, _F), W2, b2.reshape(1, _F),
    g.reshape(1, _F), bt.reshape(1, _F))


def _tc_final(h, parts, W1, b1, W2, b2, g, bt, batch_idx, Wp, bp):
  def body(h_ref, p_ref, w1_ref, b1_ref, w2_ref, b2_ref, g_ref, bt_ref,
           bi_ref, wp_ref, bp_ref, o_ref):
    h2 = h_ref[...] + p_ref[0] + p_ref[1]
    z = jnp.dot(h2, w1_ref[...], preferred_element_type=jnp.float32) + b1_ref[...]
    z = jnp.maximum(z, 0.0)
    z = jnp.dot(z, w2_ref[...], preferred_element_type=jnp.float32) + b2_ref[...]
    mean = jnp.mean(z, axis=0, keepdims=True)
    d = z - mean
    var = jnp.mean(d * d, axis=0, keepdims=True)
    scale = g_ref[...] * lax.rsqrt(var + 1e-5)
    z = jnp.maximum(d * scale + bt_ref[...], 0.0)
    onehot = (bi_ref[...] == lax.broadcasted_iota(jnp.int32, (1, _G), 1)
              ).astype(jnp.float32)
    sums = lax.dot_general(onehot, z, (((0,), (0,)), ((), ())),
                           preferred_element_type=jnp.float32)
    cnts = jnp.sum(onehot, axis=0, keepdims=True)
    pooled = sums * (1.0 / jnp.maximum(cnts, 1.0)).T
    out = jnp.dot(pooled, wp_ref[...], preferred_element_type=jnp.float32)
    o_ref[...] = jnp.maximum(out + bp_ref[...], 0.0)

  return pl.pallas_call(
      body,
      out_shape=jax.ShapeDtypeStruct((_G, _G), jnp.float32),
  )(h, parts, W1, b1.reshape(1, _F), W2, b2.reshape(1, _F),
    g.reshape(1, _F), bt.reshape(1, _F), batch_idx.reshape(_N, 1),
    Wp, bp.reshape(1, _G))


def kernel(x, edge_index, batch_idx, W1_0, b1_0, W2_0, b2_0, g_0, bt_0,
           W1_1, b1_1, W2_1, b2_1, g_1, bt_1, W1_2, b1_2, W2_2, b2_2, g_2,
           bt_2, Wp, bp):
  src = edge_index[0]
  dst = edge_index[1]
  e0 = _NSUB * _K0 * _CH
  cap1 = _NSUB * _K1 * _CH
  pad = e0 + cap1 - _E

  def mk(v, fill, k, lo, hi, padn):
    vv = jnp.concatenate([v[lo:hi], jnp.full((padn,), fill, jnp.int32)]) \
        if padn else v[lo:hi]
    return vv.reshape(_NSUB, k, _CH)

  idx0 = jnp.stack([mk(src, 0, _K0, 0, e0, 0),
                    mk(dst, _TRASH, _K0, 0, e0, 0)], axis=2)
  idx1 = jnp.stack([mk(src, 0, _K1, e0, _E, pad),
                    mk(dst, _TRASH, _K1, e0, _E, pad)], axis=2)

  h = x
  parts = _sc_aggregate(h, idx0, idx1)
  h = _tc_layer(h, parts, W1_0, b1_0, W2_0, b2_0, g_0, bt_0)
  parts = _sc_aggregate(h, idx0, idx1)
  h = _tc_layer(h, parts, W1_1, b1_1, W2_1, b2_1, g_1, bt_1)
  parts = _sc_aggregate(h, idx0, idx1)
  return _tc_final(h, parts, W1_2, b1_2, W2_2, b2_2, g_2, bt_2,
                   batch_idx, Wp, bp)
